# baseline (device time: 47810 ns/iter reference)
import os

import jax
import jax.numpy as jnp
from jax import lax
from jax.experimental import pallas as pl
from jax.experimental.pallas import tpu as pltpu

N_DEV = 8
B, SQ, SKV, E = 2, 512, 512, 768
HQ, DH = 64, 64
H_LOC = HQ // N_DEV
BLK = 64
R = B * SQ
CH = R // N_DEV


def kernel(x, Wq, K_ext, V_ext, Wo):
    h0 = lax.axis_index("i") * H_LOC
    k_shard = lax.dynamic_slice(K_ext, (0, 0, h0, 0), (B, SKV, H_LOC, DH))
    v_shard = lax.dynamic_slice(V_ext, (0, 0, h0, 0), (B, SKV, H_LOC, DH))

    def body(x_ref, wq_ref, k_ref, v_ref, wo_ref, out_ref,
             cbuf, rbuf, rs_send, rs_recv, ag_send, ag_recv):
        my = lax.axis_index("i")

        barrier_sem = pltpu.get_barrier_semaphore()
        for j in range(N_DEV):
            @pl.when(j != my)
            def _(j=j):
                pl.semaphore_signal(barrier_sem, inc=1, device_id=(j,),
                                    device_id_type=pl.DeviceIdType.MESH)
        pl.semaphore_wait(barrier_sem, N_DEV - 1)

        wq = wq_ref[...].astype(jnp.bfloat16)
        wo = wo_ref[...].astype(jnp.bfloat16)

        LOG2E = 1.4426950408889634

        def compute_batch(b):
            qb = jnp.dot(x_ref[b].astype(jnp.bfloat16), wq,
                         preferred_element_type=jnp.float32
                         ).astype(jnp.bfloat16)
            kb = k_ref[b].astype(jnp.bfloat16)
            vb = v_ref[b].astype(jnp.bfloat16)
            ctx_grp = []
            for m in range(4):
                lo, hi = BLK * m, BLK * (m + 4)
                q_m = jnp.concatenate([qb[lo:lo + BLK], qb[hi:hi + BLK]], 0)
                k_m = jnp.concatenate([kb[lo:lo + BLK], kb[hi:hi + BLK]], 0)
                v_m = jnp.concatenate([vb[lo:lo + BLK], vb[hi:hi + BLK]], 0)
                cols = []
                for h in range(H_LOC):
                    s = lax.dot_general(
                        q_m[:, h * DH:(h + 1) * DH], k_m[:, h, :],
                        (((1,), (1,)), ((), ())),
                        preferred_element_type=jnp.float32)
                    w = jnp.exp2(s * (0.125 * LOG2E))
                    denom = jnp.sum(w, axis=1, keepdims=True)
                    ctx_h = jnp.dot(w.astype(jnp.bfloat16), v_m[:, h, :],
                                    preferred_element_type=jnp.float32)
                    cols.append((ctx_h / denom).astype(jnp.bfloat16))
                ctx_grp.append(jnp.concatenate(cols, axis=1))
            ctx = jnp.concatenate(
                [ctx_grp[m][half * BLK:(half + 1) * BLK]
                 for half in range(2) for m in range(4)], axis=0)
            return jnp.dot(ctx, wo, preferred_element_type=jnp.float32)

        def rs_to(c):
            return pltpu.make_async_remote_copy(
                src_ref=cbuf.at[pl.ds(CH * c, CH)],
                dst_ref=rbuf.at[pl.ds(CH * my, CH)],
                send_sem=rs_send.at[c],
                recv_sem=rs_recv.at[my],
                device_id=(c,),
                device_id_type=pl.DeviceIdType.MESH,
            )

        def rs_from(s):
            return pltpu.make_async_remote_copy(
                src_ref=cbuf.at[pl.ds(CH * s, CH)],
                dst_ref=rbuf.at[pl.ds(CH * s, CH)],
                send_sem=rs_send.at[s],
                recv_sem=rs_recv.at[s],
                device_id=(s,),
                device_id_type=pl.DeviceIdType.MESH,
            )

        def ag_to(j):
            return pltpu.make_async_remote_copy(
                src_ref=cbuf.at[pl.ds(CH * my, CH)],
                dst_ref=cbuf.at[pl.ds(CH * my, CH)],
                send_sem=ag_send.at[j],
                recv_sem=ag_recv.at[my],
                device_id=(j,),
                device_id_type=pl.DeviceIdType.MESH,
            )

        def ag_from(s):
            return pltpu.make_async_remote_copy(
                src_ref=cbuf.at[pl.ds(CH * s, CH)],
                dst_ref=cbuf.at[pl.ds(CH * s, CH)],
                send_sem=ag_send.at[s],
                recv_sem=ag_recv.at[s],
                device_id=(s,),
                device_id_type=pl.DeviceIdType.MESH,
            )

        ablate = os.environ.get("ABLATE", "")
        for b in range(B):
            if ablate == "comm":
                part = jnp.zeros((SQ, E), jnp.float32)
            else:
                part = compute_batch(b)
            cbuf[pl.ds(b * SQ, SQ)] = part.astype(jnp.bfloat16)
            if ablate == "compute":
                out_ref[pl.ds(b * SQ, SQ)] = part
            else:
                for c in range(b * 4, b * 4 + 4):
                    @pl.when(c != my)
                    def _(c=c):
                        rs_to(c).start()
        if ablate == "compute":
            return

        rbuf[pl.ds(CH * my, CH)] = cbuf[pl.ds(CH * my, CH)]
        for s in range(N_DEV):
            @pl.when(s != my)
            def _(s=s):
                rs_from(s).wait_recv()

        red = jnp.sum(rbuf[...].reshape(N_DEV, CH, E).astype(jnp.float32),
                      axis=0)
        out_ref[pl.ds(CH * my, CH)] = red
        cbuf[pl.ds(CH * my, CH)] = red.astype(jnp.bfloat16)

        for j in range(N_DEV):
            @pl.when(j != my)
            def _(j=j):
                ag_to(j).start()

        for s in range(N_DEV):
            @pl.when(s != my)
            def _(s=s):
                ag_from(s).wait_recv()
                out_ref[pl.ds(CH * s, CH)] = (
                    cbuf[pl.ds(CH * s, CH)].astype(jnp.float32))

        for c in range(N_DEV):
            @pl.when(c != my)
            def _(c=c):
                rs_to(c).wait_send()
                ag_to(c).wait_send()

    out = pl.pallas_call(
        body,
        out_shape=jax.ShapeDtypeStruct((R, E), jnp.float32),
        in_specs=[pl.BlockSpec(memory_space=pltpu.VMEM)] * 5,
        out_specs=pl.BlockSpec(memory_space=pltpu.VMEM),
        scratch_shapes=[
            pltpu.VMEM((R, E), jnp.bfloat16),
            pltpu.VMEM((R, E), jnp.bfloat16),
            pltpu.SemaphoreType.DMA((N_DEV,)),
            pltpu.SemaphoreType.DMA((N_DEV,)),
            pltpu.SemaphoreType.DMA((N_DEV,)),
            pltpu.SemaphoreType.DMA((N_DEV,)),
        ],
        compiler_params=pltpu.CompilerParams(collective_id=0),
    )(x, Wq, k_shard, v_shard, Wo)
    return out.reshape(B, SQ, E)


# device time: 47521 ns/iter; 1.0061x vs baseline; 1.0061x over previous
import os

import jax
import jax.numpy as jnp
from jax import lax
from jax.experimental import pallas as pl
from jax.experimental.pallas import tpu as pltpu

N_DEV = 8
B, SQ, SKV, E = 2, 512, 512, 768
HQ, DH = 64, 64
H_LOC = HQ // N_DEV
BLK = 64
R = B * SQ
CH = R // N_DEV


def kernel(x, Wq, K_ext, V_ext, Wo):
    h0 = lax.axis_index("i") * H_LOC
    k_shard = lax.dynamic_slice(K_ext, (0, 0, h0, 0), (B, SKV, H_LOC, DH))
    v_shard = lax.dynamic_slice(V_ext, (0, 0, h0, 0), (B, SKV, H_LOC, DH))

    def body(x_ref, wq_ref, k_ref, v_ref, wo_ref, out_ref,
             cbuf, rbuf, rs_send, rs_recv, ag_send, ag_recv):
        my = lax.axis_index("i")

        barrier_sem = pltpu.get_barrier_semaphore()
        for j in range(N_DEV):
            @pl.when(j != my)
            def _(j=j):
                pl.semaphore_signal(barrier_sem, inc=1, device_id=(j,),
                                    device_id_type=pl.DeviceIdType.MESH)
        pl.semaphore_wait(barrier_sem, N_DEV - 1)

        wq = wq_ref[...].astype(jnp.bfloat16)
        wo = wo_ref[...].astype(jnp.bfloat16)

        rows = lax.broadcasted_iota(jnp.int32, (SQ, SKV), 0) // BLK
        cols = lax.broadcasted_iota(jnp.int32, (SQ, SKV), 1) // BLK
        live = (rows == cols) | ((cols % 4) == (rows % 4))
        LOG2E = 1.4426950408889634
        bias = jnp.where(live, jnp.float32(0.0), jnp.float32(-30.0 * LOG2E))

        QR = SQ // 2

        def compute_quarter(b, half):
            lo = half * QR
            qb = jnp.dot(x_ref[b][lo:lo + QR].astype(jnp.bfloat16), wq,
                         preferred_element_type=jnp.float32
                         ).astype(jnp.bfloat16)
            kb = k_ref[b].astype(jnp.bfloat16)
            vb = v_ref[b].astype(jnp.bfloat16)
            bias_q = bias[lo:lo + QR]
            ctx_cols = []
            for h in range(H_LOC):
                q = qb[:, h * DH:(h + 1) * DH]
                s = lax.dot_general(q, kb[:, h, :], (((1,), (1,)), ((), ())),
                                    preferred_element_type=jnp.float32)
                w = jnp.exp2(s * (0.125 * LOG2E) + bias_q)
                denom = jnp.sum(w, axis=1, keepdims=True)
                ctx_h = jnp.dot(w.astype(jnp.bfloat16), vb[:, h, :],
                                preferred_element_type=jnp.float32)
                ctx_cols.append((ctx_h / denom).astype(jnp.bfloat16))
            ctx = jnp.concatenate(ctx_cols, axis=1)
            return jnp.dot(ctx, wo, preferred_element_type=jnp.float32)

        def rs_to(c):
            return pltpu.make_async_remote_copy(
                src_ref=cbuf.at[pl.ds(CH * c, CH)],
                dst_ref=rbuf.at[pl.ds(CH * my, CH)],
                send_sem=rs_send.at[c],
                recv_sem=rs_recv.at[my],
                device_id=(c,),
                device_id_type=pl.DeviceIdType.MESH,
            )

        def rs_from(s):
            return pltpu.make_async_remote_copy(
                src_ref=cbuf.at[pl.ds(CH * s, CH)],
                dst_ref=rbuf.at[pl.ds(CH * s, CH)],
                send_sem=rs_send.at[s],
                recv_sem=rs_recv.at[s],
                device_id=(s,),
                device_id_type=pl.DeviceIdType.MESH,
            )

        def ag_to(j):
            return pltpu.make_async_remote_copy(
                src_ref=cbuf.at[pl.ds(CH * my, CH)],
                dst_ref=cbuf.at[pl.ds(CH * my, CH)],
                send_sem=ag_send.at[j],
                recv_sem=ag_recv.at[my],
                device_id=(j,),
                device_id_type=pl.DeviceIdType.MESH,
            )

        def ag_from(s):
            return pltpu.make_async_remote_copy(
                src_ref=cbuf.at[pl.ds(CH * s, CH)],
                dst_ref=cbuf.at[pl.ds(CH * s, CH)],
                send_sem=ag_send.at[s],
                recv_sem=ag_recv.at[s],
                device_id=(s,),
                device_id_type=pl.DeviceIdType.MESH,
            )

        ablate = os.environ.get("ABLATE", "")
        for quarter in range(4):
            b, half = divmod(quarter, 2)
            if ablate == "comm":
                part = jnp.zeros((QR, E), jnp.float32)
            else:
                part = compute_quarter(b, half)
            cbuf[pl.ds(quarter * QR, QR)] = part.astype(jnp.bfloat16)
            if ablate == "compute":
                out_ref[pl.ds(quarter * QR, QR)] = part
            else:
                for c in range(quarter * 2, quarter * 2 + 2):
                    @pl.when(c != my)
                    def _(c=c):
                        rs_to(c).start()
        if ablate == "compute":
            return

        rbuf[pl.ds(CH * my, CH)] = cbuf[pl.ds(CH * my, CH)]
        for s in range(N_DEV):
            @pl.when(s != my)
            def _(s=s):
                rs_from(s).wait_recv()

        red = jnp.sum(rbuf[...].reshape(N_DEV, CH, E).astype(jnp.float32),
                      axis=0)
        out_ref[pl.ds(CH * my, CH)] = red
        cbuf[pl.ds(CH * my, CH)] = red.astype(jnp.bfloat16)

        for j in range(N_DEV):
            @pl.when(j != my)
            def _(j=j):
                ag_to(j).start()

        for s in range(N_DEV):
            @pl.when(s != my)
            def _(s=s):
                ag_from(s).wait_recv()
                out_ref[pl.ds(CH * s, CH)] = (
                    cbuf[pl.ds(CH * s, CH)].astype(jnp.float32))

        for c in range(N_DEV):
            @pl.when(c != my)
            def _(c=c):
                rs_to(c).wait_send()
                ag_to(c).wait_send()

    out = pl.pallas_call(
        body,
        out_shape=jax.ShapeDtypeStruct((R, E), jnp.float32),
        in_specs=[pl.BlockSpec(memory_space=pltpu.VMEM)] * 5,
        out_specs=pl.BlockSpec(memory_space=pltpu.VMEM),
        scratch_shapes=[
            pltpu.VMEM((R, E), jnp.bfloat16),
            pltpu.VMEM((R, E), jnp.bfloat16),
            pltpu.SemaphoreType.DMA((N_DEV,)),
            pltpu.SemaphoreType.DMA((N_DEV,)),
            pltpu.SemaphoreType.DMA((N_DEV,)),
            pltpu.SemaphoreType.DMA((N_DEV,)),
        ],
        compiler_params=pltpu.CompilerParams(collective_id=0),
    )(x, Wq, k_shard, v_shard, Wo)
    return out.reshape(B, SQ, E)


# device time: 42178 ns/iter; 1.1335x vs baseline; 1.1267x over previous
import os

import jax
import jax.numpy as jnp
from jax import lax
from jax.experimental import pallas as pl
from jax.experimental.pallas import tpu as pltpu

N_DEV = 8
B, SQ, SKV, E = 2, 512, 512, 768
HQ, DH = 64, 64
H_LOC = HQ // N_DEV
BLK = 64
R = B * SQ
CH = R // N_DEV


def kernel(x, Wq, K_ext, V_ext, Wo):
    h0 = lax.axis_index("i") * H_LOC
    k_shard = lax.dynamic_slice(K_ext, (0, 0, h0, 0), (B, SKV, H_LOC, DH))
    v_shard = lax.dynamic_slice(V_ext, (0, 0, h0, 0), (B, SKV, H_LOC, DH))

    def body(x_ref, wq_ref, k_ref, v_ref, wo_ref, out_ref,
             cbuf, rbuf, rs_send, rs_recv, ag_send, ag_recv):
        my = lax.axis_index("i")

        barrier_sem = pltpu.get_barrier_semaphore()
        for j in range(N_DEV):
            @pl.when(j != my)
            def _(j=j):
                pl.semaphore_signal(barrier_sem, inc=1, device_id=(j,),
                                    device_id_type=pl.DeviceIdType.MESH)
        pl.semaphore_wait(barrier_sem, N_DEV - 1)

        wq = wq_ref[...].astype(jnp.bfloat16)
        wo = wo_ref[...].astype(jnp.bfloat16)

        rows = lax.broadcasted_iota(jnp.int32, (SQ, SKV), 0) // BLK
        cols = lax.broadcasted_iota(jnp.int32, (SQ, SKV), 1) // BLK
        live = (rows == cols) | ((cols % 4) == (rows % 4))
        LOG2E = 1.4426950408889634
        bias = jnp.where(live, jnp.float32(0.0), jnp.float32(-30.0 * LOG2E))

        def compute_batch(b):
            qb = jnp.dot(x_ref[b].astype(jnp.bfloat16), wq,
                         preferred_element_type=jnp.float32
                         ).astype(jnp.bfloat16)
            kb = k_ref[b].astype(jnp.bfloat16)
            vb = v_ref[b].astype(jnp.bfloat16)
            ctx_cols = []
            for h in range(H_LOC):
                q = qb[:, h * DH:(h + 1) * DH]
                s = lax.dot_general(q, kb[:, h, :], (((1,), (1,)), ((), ())),
                                    preferred_element_type=jnp.float32)
                w = jnp.exp2(s * (0.125 * LOG2E) + bias)
                denom = jnp.sum(w, axis=1, keepdims=True)
                ctx_h = jnp.dot(w.astype(jnp.bfloat16), vb[:, h, :],
                                preferred_element_type=jnp.float32)
                ctx_cols.append((ctx_h / denom).astype(jnp.bfloat16))
            ctx = jnp.concatenate(ctx_cols, axis=1)
            return jnp.dot(ctx, wo, preferred_element_type=jnp.float32)

        def rs_to(c):
            return pltpu.make_async_remote_copy(
                src_ref=cbuf.at[pl.ds(CH * c, CH)],
                dst_ref=rbuf.at[pl.ds(CH * my, CH)],
                send_sem=rs_send.at[c],
                recv_sem=rs_recv.at[my],
                device_id=(c,),
                device_id_type=pl.DeviceIdType.MESH,
            )

        def rs_from(s):
            return pltpu.make_async_remote_copy(
                src_ref=cbuf.at[pl.ds(CH * s, CH)],
                dst_ref=rbuf.at[pl.ds(CH * s, CH)],
                send_sem=rs_send.at[s],
                recv_sem=rs_recv.at[s],
                device_id=(s,),
                device_id_type=pl.DeviceIdType.MESH,
            )

        def ag_to(j):
            return pltpu.make_async_remote_copy(
                src_ref=cbuf.at[pl.ds(CH * my, CH)],
                dst_ref=cbuf.at[pl.ds(CH * my, CH)],
                send_sem=ag_send.at[j],
                recv_sem=ag_recv.at[my],
                device_id=(j,),
                device_id_type=pl.DeviceIdType.MESH,
            )

        def ag_from(s):
            return pltpu.make_async_remote_copy(
                src_ref=cbuf.at[pl.ds(CH * s, CH)],
                dst_ref=cbuf.at[pl.ds(CH * s, CH)],
                send_sem=ag_send.at[s],
                recv_sem=ag_recv.at[s],
                device_id=(s,),
                device_id_type=pl.DeviceIdType.MESH,
            )

        ablate = os.environ.get("ABLATE", "")
        for b in range(B):
            if ablate == "comm":
                part = jnp.zeros((SQ, E), jnp.float32)
            else:
                part = compute_batch(b)
            cbuf[pl.ds(b * SQ, SQ)] = part.astype(jnp.bfloat16)
            if ablate == "compute":
                out_ref[pl.ds(b * SQ, SQ)] = part
            else:
                for c in range(b * 4, b * 4 + 4):
                    @pl.when(c != my)
                    def _(c=c):
                        rs_to(c).start()
        if ablate == "compute":
            return

        rbuf[pl.ds(CH * my, CH)] = cbuf[pl.ds(CH * my, CH)]
        for s in range(N_DEV):
            @pl.when(s != my)
            def _(s=s):
                rs_from(s).wait_recv()

        red = jnp.sum(rbuf[...].reshape(N_DEV, CH, E).astype(jnp.float32),
                      axis=0)
        out_ref[pl.ds(CH * my, CH)] = red
        cbuf[pl.ds(CH * my, CH)] = red.astype(jnp.bfloat16)

        for j in range(N_DEV):
            @pl.when(j != my)
            def _(j=j):
                ag_to(j).start()

        for s in range(N_DEV):
            @pl.when(s != my)
            def _(s=s):
                ag_from(s).wait_recv()
                out_ref[pl.ds(CH * s, CH)] = (
                    cbuf[pl.ds(CH * s, CH)].astype(jnp.float32))

        for c in range(N_DEV):
            @pl.when(c != my)
            def _(c=c):
                rs_to(c).wait_send()
                ag_to(c).wait_send()

    out = pl.pallas_call(
        body,
        out_shape=jax.ShapeDtypeStruct((R, E), jnp.float32),
        in_specs=[pl.BlockSpec(memory_space=pltpu.VMEM)] * 5,
        out_specs=pl.BlockSpec(memory_space=pltpu.VMEM),
        scratch_shapes=[
            pltpu.VMEM((R, E), jnp.bfloat16),
            pltpu.VMEM((R, E), jnp.bfloat16),
            pltpu.SemaphoreType.DMA((N_DEV,)),
            pltpu.SemaphoreType.DMA((N_DEV,)),
            pltpu.SemaphoreType.DMA((N_DEV,)),
            pltpu.SemaphoreType.DMA((N_DEV,)),
        ],
        compiler_params=pltpu.CompilerParams(collective_id=0),
    )(x, Wq, k_shard, v_shard, Wo)
    return out.reshape(B, SQ, E)
